# baseline (speedup 1.0000x reference)
import functools

import jax
import jax.numpy as jnp
from jax import lax
from jax.experimental import pallas as pl
from jax.experimental.pallas import tpu as pltpu
from jax.experimental.pallas import tpu_sc as plsc

_SR = 44100
_WT_LEN = 512
_N_WT = 64
_TB = 1024
_NSC = 32


def _block_cumsum(x, tb):
    sh = 1
    while sh < tb:
        x = x + jnp.pad(x[:, :-sh], ((0, 0), (sh, 0)))
        sh *= 2
    return x


def _k1_body(pitch_ref, y_ref, il_ref, alpha_ref, idx_ref, w_ref, carry_ref):
    g = pl.program_id(0)
    b, tb = pitch_ref.shape

    @pl.when(g == 0)
    def _init():
        carry_ref[...] = jnp.zeros_like(carry_ref)

    inc = pitch_ref[...] * (_WT_LEN / _SR)
    cs = _block_cumsum(inc, tb)
    carry = carry_ref[:b, :1]
    index = cs + carry - inc[0:1, :]
    carry_ref[:b, :1] = carry + cs[:, -1:]

    index = jnp.mod(index, float(_WT_LEN))
    low = jnp.floor(index)
    alpha = index - low
    il = jnp.clip(low.astype(jnp.int32), 0, _WT_LEN - 1)
    ih = jnp.where(il == _WT_LEN - 1, 0, il + 1)

    il_ref[...] = il
    alpha_ref[...] = alpha
    idx_ref[...] = jnp.concatenate([il * _N_WT, ih * _N_WT], axis=0)
    w = y_ref[...] * (1.0 / b)
    w_ref[...] = jnp.concatenate([w * (1.0 - alpha), w * alpha], axis=0)


def _sc_att_body(tcc, nck, tpad, tbl_hbm, idx_hbm, w_hbm, att_hbm,
                 tbl_v, idx_v, w_v, acc_v, sem_in, sem_out):
    wid = lax.axis_index("s") * 2 + lax.axis_index("c")
    pltpu.sync_copy(tbl_hbm, tbl_v)
    base_w = wid * (tcc * nck)

    def stage(ci, buf):
        base = base_w + ci * tcc
        cps = []
        for j in range(8):
            cps.append(pltpu.async_copy(
                idx_hbm.at[pl.ds(j * tpad + base, tcc)],
                idx_v.at[buf, pl.ds(j * tcc, tcc)], sem_in))
            cps.append(pltpu.async_copy(
                w_hbm.at[pl.ds(j * tpad + base, tcc)],
                w_v.at[buf, pl.ds(j * tcc, tcc)], sem_in))
        return cps

    in_flight = stage(0, 0)
    out_flight = []
    for ci in range(nck):
        buf = ci % 2
        for c in in_flight:
            c.wait()
        in_flight = stage(ci + 1, 1 - buf) if ci + 1 < nck else []
        for c in out_flight:
            c.wait()
        out_flight = []

        def group(gi, c, buf=buf):
            t0 = gi * 16
            idxs = [idx_v[buf, pl.ds(j * tcc + t0, 16)] for j in range(8)]
            ws = [w_v[buf, pl.ds(j * tcc + t0, 16)] for j in range(8)]

            @plsc.parallel_loop(0, _N_WT, unroll=2)
            def _fl(fl):
                gs = [plsc.load_gather(tbl_v, [idxs[j] + fl]) for j in range(8)]
                a01 = ws[0] * gs[0] + ws[1] * gs[1]
                a23 = ws[2] * gs[2] + ws[3] * gs[3]
                a45 = ws[4] * gs[4] + ws[5] * gs[5]
                a67 = ws[6] * gs[6] + ws[7] * gs[7]
                acc_v[buf, fl, pl.ds(t0, 16)] = (a01 + a23) + (a45 + a67)

            return c

        lax.fori_loop(0, tcc // 16, group, 0)
        base = base_w + ci * tcc
        out_flight = [pltpu.async_copy(
            acc_v.at[buf], att_hbm.at[:, pl.ds(base, tcc)], sem_out)]
    for c in out_flight:
        c.wait()


def _k2_body(main_ref, left_ref, right_ref, il_ref, alpha_ref, amp_ref,
             wt_ref, cw_ref, out_ref):
    g = pl.program_id(0)
    b, tb = il_ref.shape

    lh = jnp.where(g == 0, 0.0, left_ref[:, -2:])
    window = jnp.concatenate([lh, main_ref[...], right_ref[:, :2]], axis=1)
    att = jnp.full((_N_WT, tb), cw_ref[5], jnp.float32)
    for k in range(5):
        att = att + cw_ref[k] * window[:, k : k + tb]

    mx = jnp.max(att, axis=0, keepdims=True)
    e = jnp.exp(att - mx)
    s = e / jnp.sum(e, axis=0, keepdims=True)

    il = il_ref[...]
    alpha = alpha_ref[...]
    ih = jnp.where(il == _WT_LEN - 1, 0, il + 1)
    iota = lax.broadcasted_iota(jnp.int32, (_WT_LEN, tb), 0)
    rows = []
    for bb in range(b):
        m = jnp.where(iota == il[bb : bb + 1, :], 1.0 - alpha[bb : bb + 1, :], 0.0)
        m = m + jnp.where(iota == ih[bb : bb + 1, :], alpha[bb : bb + 1, :], 0.0)
        mixed = jnp.dot(wt_ref[...], m, preferred_element_type=jnp.float32)
        rows.append(jnp.sum(s * mixed, axis=0, keepdims=True))
    out_ref[...] = jnp.concatenate(rows, axis=0) * amp_ref[...]


def _pick_tcc(tc):
    best = 128
    for k in range(128, 1025, 128):
        if tc % k == 0:
            best = k
    return best


@jax.jit
def _run(pitch, amplitude, y, WT, conv_w, conv_b):
    b, t = pitch.shape
    g = pl.cdiv(t, _TB)
    g += -g % (_NSC * 128 // _TB if _TB < _NSC * 128 else 1)
    tpad = g * _TB
    pad = tpad - t
    pitch_p = jnp.pad(pitch, ((0, 0), (0, pad)))
    y_p = jnp.pad(y, ((0, 0), (0, pad)))
    amp_p = jnp.pad(amplitude[..., 0], ((0, 0), (0, pad)))
    cw = jnp.concatenate([conv_w.reshape(5), conv_b.reshape(1),
                          jnp.zeros((2,), jnp.float32)])
    tbl = WT.T.reshape(-1)

    il, alpha, idx_all, w_all = pl.pallas_call(
        _k1_body,
        grid=(g,),
        in_specs=[
            pl.BlockSpec((b, _TB), lambda i: (0, i)),
            pl.BlockSpec((b, _TB), lambda i: (0, i)),
        ],
        out_specs=[
            pl.BlockSpec((b, _TB), lambda i: (0, i)),
            pl.BlockSpec((b, _TB), lambda i: (0, i)),
            pl.BlockSpec((2 * b, _TB), lambda i: (0, i)),
            pl.BlockSpec((2 * b, _TB), lambda i: (0, i)),
        ],
        out_shape=[
            jax.ShapeDtypeStruct((b, tpad), jnp.int32),
            jax.ShapeDtypeStruct((b, tpad), jnp.float32),
            jax.ShapeDtypeStruct((2 * b, tpad), jnp.int32),
            jax.ShapeDtypeStruct((2 * b, tpad), jnp.float32),
        ],
        scratch_shapes=[pltpu.VMEM((8, 128), jnp.float32)],
    )(pitch_p, y_p)

    tc = tpad // _NSC
    tcc = _pick_tcc(tc)
    nck = tc // tcc
    att = pl.kernel(
        functools.partial(_sc_att_body, tcc, nck, tpad),
        out_type=jax.ShapeDtypeStruct((_N_WT, tpad), jnp.float32),
        mesh=plsc.VectorSubcoreMesh(core_axis_name="c", subcore_axis_name="s"),
        compiler_params=pltpu.CompilerParams(needs_layout_passes=False),
        scratch_types=[
            pltpu.VMEM((_WT_LEN * _N_WT,), jnp.float32),
            pltpu.VMEM((2, 8 * tcc), jnp.int32),
            pltpu.VMEM((2, 8 * tcc), jnp.float32),
            pltpu.VMEM((2, _N_WT, tcc), jnp.float32),
            pltpu.SemaphoreType.DMA,
            pltpu.SemaphoreType.DMA,
        ],
    )(tbl, idx_all.reshape(-1), w_all.reshape(-1))

    nhb = _TB // 128
    last_hb = tpad // 128 - 1
    out = pl.pallas_call(
        _k2_body,
        grid=(g,),
        in_specs=[
            pl.BlockSpec((_N_WT, _TB), lambda i: (0, i)),
            pl.BlockSpec((_N_WT, 128), lambda i: (0, jnp.maximum(i * nhb - 1, 0))),
            pl.BlockSpec((_N_WT, 128), lambda i: (0, jnp.minimum((i + 1) * nhb, last_hb))),
            pl.BlockSpec((b, _TB), lambda i: (0, i)),
            pl.BlockSpec((b, _TB), lambda i: (0, i)),
            pl.BlockSpec((b, _TB), lambda i: (0, i)),
            pl.BlockSpec((_N_WT, _WT_LEN), lambda i: (0, 0)),
            pl.BlockSpec(memory_space=pltpu.SMEM),
        ],
        out_specs=pl.BlockSpec((b, _TB), lambda i: (0, i)),
        out_shape=jax.ShapeDtypeStruct((b, tpad), jnp.float32),
    )(att, att, att, il, alpha, amp_p, WT, cw)

    return out[:, :t, None]


def kernel(pitch, amplitude, y, WT, conv_w, conv_b, duration_secs):
    return _run(pitch, amplitude, y, WT, conv_w, conv_b)


# speedup vs baseline: 1.0947x; 1.0947x over previous
import functools

import jax
import jax.numpy as jnp
from jax import lax
from jax.experimental import pallas as pl
from jax.experimental.pallas import tpu as pltpu
from jax.experimental.pallas import tpu_sc as plsc

_SR = 44100
_WT_LEN = 512
_N_WT = 64
_TB = 2048
_NSC = 32


def _block_cumsum(x, tb):
    sh = 1
    while sh < tb:
        x = x + jnp.pad(x[:, :-sh], ((0, 0), (sh, 0)))
        sh *= 2
    return x


def _k1_body(pitch_ref, y_ref, il_ref, alpha_ref, idx_ref, w_ref, carry_ref):
    g = pl.program_id(0)
    b, tb = pitch_ref.shape

    @pl.when(g == 0)
    def _init():
        carry_ref[...] = jnp.zeros_like(carry_ref)

    inc = pitch_ref[...] * (_WT_LEN / _SR)
    cs = _block_cumsum(inc, tb)
    carry = carry_ref[:b, :1]
    index = cs + carry - inc[0:1, :]
    carry_ref[:b, :1] = carry + cs[:, -1:]

    index = jnp.mod(index, float(_WT_LEN))
    low = jnp.floor(index)
    alpha = index - low
    il = jnp.clip(low.astype(jnp.int32), 0, _WT_LEN - 1)
    ih = jnp.where(il == _WT_LEN - 1, 0, il + 1)

    il_ref[...] = il
    alpha_ref[...] = alpha
    idx_ref[...] = jnp.concatenate([il * _N_WT, ih * _N_WT], axis=0)
    w = y_ref[...] * (1.0 / b)
    w_ref[...] = jnp.concatenate([w * (1.0 - alpha), w * alpha], axis=0)


def _sc_att_body(tcc, nck, tpad, tbl_hbm, idx_hbm, w_hbm, att_hbm,
                 tbl_v, idx_v, w_v, acc_v, sem_in, sem_out):
    wid = lax.axis_index("s") * 2 + lax.axis_index("c")
    pltpu.sync_copy(tbl_hbm, tbl_v)
    base_w = wid * (tcc * nck)

    def stage(ci, buf):
        base = base_w + ci * tcc
        cps = []
        for j in range(8):
            cps.append(pltpu.async_copy(
                idx_hbm.at[pl.ds(j * tpad + base, tcc)],
                idx_v.at[buf, pl.ds(j * tcc, tcc)], sem_in))
            cps.append(pltpu.async_copy(
                w_hbm.at[pl.ds(j * tpad + base, tcc)],
                w_v.at[buf, pl.ds(j * tcc, tcc)], sem_in))
        return cps

    in_flight = stage(0, 0)
    out_flight = []
    for ci in range(nck):
        buf = ci % 2
        for c in in_flight:
            c.wait()
        in_flight = stage(ci + 1, 1 - buf) if ci + 1 < nck else []
        for c in out_flight:
            c.wait()
        out_flight = []

        def group(gi, c, buf=buf):
            t0 = gi * 16
            idxs = [idx_v[buf, pl.ds(j * tcc + t0, 16)] for j in range(8)]
            ws = [w_v[buf, pl.ds(j * tcc + t0, 16)] for j in range(8)]

            @plsc.parallel_loop(0, _N_WT, unroll=2)
            def _fl(fl):
                gs = [plsc.load_gather(tbl_v, [idxs[j] + fl]) for j in range(8)]
                a01 = ws[0] * gs[0] + ws[1] * gs[1]
                a23 = ws[2] * gs[2] + ws[3] * gs[3]
                a45 = ws[4] * gs[4] + ws[5] * gs[5]
                a67 = ws[6] * gs[6] + ws[7] * gs[7]
                acc_v[buf, fl, pl.ds(t0, 16)] = (a01 + a23) + (a45 + a67)

            return c

        lax.fori_loop(0, tcc // 16, group, 0)
        base = base_w + ci * tcc
        out_flight = [pltpu.async_copy(
            acc_v.at[buf], att_hbm.at[:, pl.ds(base, tcc)], sem_out)]
    for c in out_flight:
        c.wait()


def _k2_body(main_ref, left_ref, right_ref, il_ref, alpha_ref, amp_ref,
             wt_ref, cw_ref, out_ref):
    g = pl.program_id(0)
    b, tb = il_ref.shape

    lh = jnp.where(g == 0, 0.0, left_ref[:, -2:])
    window = jnp.concatenate([lh, main_ref[...], right_ref[:, :2]], axis=1)
    att = jnp.full((_N_WT, tb), cw_ref[5], jnp.float32)
    for k in range(5):
        att = att + cw_ref[k] * window[:, k : k + tb]

    mx = jnp.max(att, axis=0, keepdims=True)
    e = jnp.exp(att - mx)
    s = e / jnp.sum(e, axis=0, keepdims=True)

    il = il_ref[...]
    alpha = alpha_ref[...]
    ih = jnp.where(il == _WT_LEN - 1, 0, il + 1)
    iota = lax.broadcasted_iota(jnp.int32, (_WT_LEN, tb), 0)
    rows = []
    for bb in range(b):
        m = jnp.where(iota == il[bb : bb + 1, :], 1.0 - alpha[bb : bb + 1, :], 0.0)
        m = m + jnp.where(iota == ih[bb : bb + 1, :], alpha[bb : bb + 1, :], 0.0)
        mixed = jnp.dot(wt_ref[...], m, preferred_element_type=jnp.float32)
        rows.append(jnp.sum(s * mixed, axis=0, keepdims=True))
    out_ref[...] = jnp.concatenate(rows, axis=0) * amp_ref[...]


def _pick_tcc(tc):
    best = 128
    for k in range(128, 1025, 128):
        if tc % k == 0:
            best = k
    return best


@jax.jit
def _run(pitch, amplitude, y, WT, conv_w, conv_b):
    b, t = pitch.shape
    g = pl.cdiv(t, _TB)
    g += -g % (_NSC * 128 // _TB if _TB < _NSC * 128 else 1)
    tpad = g * _TB
    pad = tpad - t
    pitch_p = jnp.pad(pitch, ((0, 0), (0, pad)))
    y_p = jnp.pad(y, ((0, 0), (0, pad)))
    amp_p = jnp.pad(amplitude[..., 0], ((0, 0), (0, pad)))
    cw = jnp.concatenate([conv_w.reshape(5), conv_b.reshape(1),
                          jnp.zeros((2,), jnp.float32)])
    tbl = WT.T.reshape(-1)

    il, alpha, idx_all, w_all = pl.pallas_call(
        _k1_body,
        grid=(g,),
        in_specs=[
            pl.BlockSpec((b, _TB), lambda i: (0, i)),
            pl.BlockSpec((b, _TB), lambda i: (0, i)),
        ],
        out_specs=[
            pl.BlockSpec((b, _TB), lambda i: (0, i)),
            pl.BlockSpec((b, _TB), lambda i: (0, i)),
            pl.BlockSpec((2 * b, _TB), lambda i: (0, i)),
            pl.BlockSpec((2 * b, _TB), lambda i: (0, i)),
        ],
        out_shape=[
            jax.ShapeDtypeStruct((b, tpad), jnp.int32),
            jax.ShapeDtypeStruct((b, tpad), jnp.float32),
            jax.ShapeDtypeStruct((2 * b, tpad), jnp.int32),
            jax.ShapeDtypeStruct((2 * b, tpad), jnp.float32),
        ],
        scratch_shapes=[pltpu.VMEM((8, 128), jnp.float32)],
    )(pitch_p, y_p)

    tc = tpad // _NSC
    tcc = _pick_tcc(tc)
    nck = tc // tcc
    att = pl.kernel(
        functools.partial(_sc_att_body, tcc, nck, tpad),
        out_type=jax.ShapeDtypeStruct((_N_WT, tpad), jnp.float32),
        mesh=plsc.VectorSubcoreMesh(core_axis_name="c", subcore_axis_name="s"),
        compiler_params=pltpu.CompilerParams(needs_layout_passes=False),
        scratch_types=[
            pltpu.VMEM((_WT_LEN * _N_WT,), jnp.float32),
            pltpu.VMEM((2, 8 * tcc), jnp.int32),
            pltpu.VMEM((2, 8 * tcc), jnp.float32),
            pltpu.VMEM((2, _N_WT, tcc), jnp.float32),
            pltpu.SemaphoreType.DMA,
            pltpu.SemaphoreType.DMA,
        ],
    )(tbl, idx_all.reshape(-1), w_all.reshape(-1))

    nhb = _TB // 128
    last_hb = tpad // 128 - 1
    out = pl.pallas_call(
        _k2_body,
        grid=(g,),
        in_specs=[
            pl.BlockSpec((_N_WT, _TB), lambda i: (0, i)),
            pl.BlockSpec((_N_WT, 128), lambda i: (0, jnp.maximum(i * nhb - 1, 0))),
            pl.BlockSpec((_N_WT, 128), lambda i: (0, jnp.minimum((i + 1) * nhb, last_hb))),
            pl.BlockSpec((b, _TB), lambda i: (0, i)),
            pl.BlockSpec((b, _TB), lambda i: (0, i)),
            pl.BlockSpec((b, _TB), lambda i: (0, i)),
            pl.BlockSpec((_N_WT, _WT_LEN), lambda i: (0, 0)),
            pl.BlockSpec(memory_space=pltpu.SMEM),
        ],
        out_specs=pl.BlockSpec((b, _TB), lambda i: (0, i)),
        out_shape=jax.ShapeDtypeStruct((b, tpad), jnp.float32),
    )(att, att, att, il, alpha, amp_p, WT, cw)

    return out[:, :t, None]


def kernel(pitch, amplitude, y, WT, conv_w, conv_b, duration_secs):
    return _run(pitch, amplitude, y, WT, conv_w, conv_b)
